# Initial kernel scaffold; baseline (speedup 1.0000x reference)
import jax
import jax.numpy as jnp
from jax.experimental import pallas as pl


def kernel(x, src1, dst1, src2, dst2, src3, dst3, n_dst1, n_dst2, n_dst3, W_self1, W_neigh1, b1, W_self2, W_neigh2, b2, W_self3, W_neigh3, b3):
    raise NotImplementedError("write your pallas kernel here")


# speedup vs baseline: 3.7361x; 3.7361x over previous
import functools

import jax
import jax.numpy as jnp
from jax import lax
from jax.experimental import pallas as pl
from jax.experimental.pallas import tpu as pltpu
from jax.experimental.pallas import tpu_sc as plsc

NC = 2
NS = 16
LANES = 16


def _sc_mean_agg(h2, src, dst, n_dst, feat_half, chunk):
    E = src.shape[0]
    e_tile = E // NS
    n_chunks = e_tile // chunk
    stripe = -(-n_dst // NS)
    stripe += (-stripe) % 8
    n_pad = stripe * NS
    assert e_tile % chunk == 0 and chunk % 8 == 0

    mesh = plsc.VectorSubcoreMesh(core_axis_name="c", subcore_axis_name="s",
                                  num_cores=NC, num_subcores=NS)

    zeros_f = jnp.zeros((n_pad, feat_half), jnp.float32)
    zeros_c = jnp.zeros((n_pad, 16), jnp.float32)
    ones_c = jnp.ones((chunk, 16), jnp.float32)

    @functools.partial(
        pl.kernel,
        out_type=(jax.ShapeDtypeStruct((NC, n_pad, feat_half), jnp.float32),
                  jax.ShapeDtypeStruct((NC, n_pad, 16), jnp.float32)),
        mesh=mesh,
        scratch_types=[
            pltpu.VMEM((chunk,), jnp.int32),
            pltpu.VMEM((chunk,), jnp.int32),
            pltpu.VMEM((chunk,), jnp.int32),
            pltpu.VMEM((chunk, feat_half), jnp.float32),
            pltpu.VMEM((chunk, 16), jnp.float32),
            pltpu.VMEM_SHARED((n_pad, feat_half), jnp.float32),
            pltpu.VMEM_SHARED((n_pad, 16), jnp.float32),
            pltpu.SemaphoreType.DMA,
        ],
        compiler_params=pltpu.CompilerParams(use_tc_tiling_on_sc=False),
    )
    def k(h2_hbm, src_hbm, dst_hbm, zf_hbm, zc_hbm, ones_hbm,
          agg_out, cnt_out,
          src_v, dst_v, idx_v, rows_v, ones_v, acc_sh, cnt_sh, sem):
        cid = lax.axis_index("c")
        sid = lax.axis_index("s")
        r0 = sid * stripe
        pltpu.sync_copy(zf_hbm.at[pl.ds(r0, stripe)],
                        acc_sh.at[pl.ds(r0, stripe)])
        pltpu.sync_copy(zc_hbm.at[pl.ds(r0, stripe)],
                        cnt_sh.at[pl.ds(r0, stripe)])
        pltpu.sync_copy(ones_hbm, ones_v)
        plsc.subcore_barrier()

        base0 = sid * e_tile
        half = n_chunks // 2

        def body(g, carry):
            base = base0 + g * chunk
            pltpu.sync_copy(src_hbm.at[pl.ds(base, chunk)], src_v)
            pltpu.sync_copy(dst_hbm.at[pl.ds(base, chunk)], dst_v)
            for i in range(chunk // LANES):
                sl = pl.ds(i * LANES, LANES)
                idx_v[sl] = src_v[sl] * 2 + cid
            pltpu.async_copy(h2_hbm.at[idx_v], rows_v, sem).wait()
            pltpu.sync_copy(rows_v, acc_sh.at[dst_v], add=True)

            @pl.when((g < half) == (cid == 0))
            def _():
                pltpu.sync_copy(ones_v, cnt_sh.at[dst_v], add=True)
            return carry

        lax.fori_loop(0, n_chunks, body, 0)
        plsc.subcore_barrier()
        pltpu.sync_copy(acc_sh.at[pl.ds(r0, stripe)],
                        agg_out.at[cid, pl.ds(r0, stripe)])
        pltpu.sync_copy(cnt_sh.at[pl.ds(r0, stripe)],
                        cnt_out.at[cid, pl.ds(r0, stripe)])

    return k(h2, src, dst, zeros_f, zeros_c, ones_c)


def _tc_sage(h_dst, agg, cnt, w_self, w_neigh, b, relu, bm):
    N, F = h_dst.shape
    fh = agg.shape[2]
    H = w_self.shape[1]

    def body(hd_ref, a_ref, c_ref, ws_ref, wn_ref, b_ref, o_ref):
        c = jnp.maximum(c_ref[0, :, 0:1] + c_ref[1, :, 0:1], 1.0)
        acc = jnp.dot(hd_ref[...], ws_ref[...],
                      preferred_element_type=jnp.float32)
        acc += jnp.dot(a_ref[0] / c, wn_ref[0],
                       preferred_element_type=jnp.float32)
        acc += jnp.dot(a_ref[1] / c, wn_ref[1],
                       preferred_element_type=jnp.float32)
        acc += b_ref[...]
        if relu:
            acc = jnp.maximum(acc, 0.0)
        o_ref[...] = acc

    return pl.pallas_call(
        body,
        grid=(N // bm,),
        in_specs=[
            pl.BlockSpec((bm, F), lambda i: (i, 0)),
            pl.BlockSpec((2, bm, fh), lambda i: (0, i, 0)),
            pl.BlockSpec((2, bm, 16), lambda i: (0, i, 0)),
            pl.BlockSpec((F, H), lambda i: (0, 0)),
            pl.BlockSpec((2, fh, H), lambda i: (0, 0, 0)),
            pl.BlockSpec((1, H), lambda i: (0, 0)),
        ],
        out_specs=pl.BlockSpec((bm, H), lambda i: (i, 0)),
        out_shape=jax.ShapeDtypeStruct((N, H), jnp.float32),
    )(h_dst, agg, cnt, w_self, w_neigh, b)


def kernel(x, src1, dst1, src2, dst2, src3, dst3, n_dst1, n_dst2, n_dst3,
           W_self1, W_neigh1, b1, W_self2, W_neigh2, b2,
           W_self3, W_neigh3, b3):
    N1, N2, N3 = 20000, 8000, 4096
    zero = ((jnp.asarray(n_dst1) - N1)
            + (jnp.asarray(n_dst2) - N2)
            + (jnp.asarray(n_dst3) - N3)).astype(x.dtype)

    s1, d1 = src1.astype(jnp.int32), dst1.astype(jnp.int32)
    s2, d2 = src2.astype(jnp.int32), dst2.astype(jnp.int32)
    s3, d3 = src3.astype(jnp.int32), dst3.astype(jnp.int32)

    agg1, cnt1 = _sc_mean_agg(x.reshape(-1, 64), s1, d1, N1, 64, chunk=80)
    h1 = _tc_sage(x[:N1], agg1, cnt1, W_self1, W_neigh1.reshape(2, 64, 256),
                  b1.reshape(1, 256), relu=True, bm=400)

    agg2, cnt2 = _sc_mean_agg(h1.reshape(-1, 128), s2, d2, N2, 128, chunk=80)
    h2 = _tc_sage(h1[:N2], agg2, cnt2, W_self2, W_neigh2.reshape(2, 128, 256),
                  b2.reshape(1, 256), relu=True, bm=400)

    agg3, cnt3 = _sc_mean_agg(h2.reshape(-1, 128), s3, d3, N3, 128, chunk=128)
    h3 = _tc_sage(h2[:N3], agg3, cnt3, W_self3, W_neigh3.reshape(2, 128, 256),
                  (b3 + zero).reshape(1, 256), relu=False, bm=512)
    return h3


# speedup vs baseline: 5.7587x; 1.5413x over previous
import functools

import jax
import jax.numpy as jnp
from jax import lax
from jax.experimental import pallas as pl
from jax.experimental.pallas import tpu as pltpu
from jax.experimental.pallas import tpu_sc as plsc

NC = 2
NS = 16
LANES = 16


def _sc_mean_agg(h2, ids, n_dst, fw, chunk, sb):
    n_chunks = ids.shape[1]
    n_sb = n_chunks // sb
    stripe = -(-n_dst // NS)
    stripe += (-stripe) % 8
    n_pad = stripe * NS
    assert n_chunks % sb == 0 and sb % 2 == 0 and chunk % 8 == 0
    assert chunk <= 128

    mesh = plsc.VectorSubcoreMesh(core_axis_name="c", subcore_axis_name="s",
                                  num_cores=NC, num_subcores=NS)
    zeros_f = jnp.zeros((n_pad, fw), jnp.float32)

    @functools.partial(
        pl.kernel,
        out_type=jax.ShapeDtypeStruct((NC, n_pad, fw), jnp.float32),
        mesh=mesh,
        scratch_types=[
            pltpu.VMEM((sb, 2, chunk), jnp.int32),
            pltpu.VMEM((chunk, fw), jnp.float32),
            pltpu.VMEM((chunk, fw), jnp.float32),
            pltpu.VMEM_SHARED((n_pad, fw), jnp.float32),
            pltpu.SemaphoreType.DMA,
            pltpu.SemaphoreType.DMA,
        ],
        compiler_params=pltpu.CompilerParams(use_tc_tiling_on_sc=False),
    )
    def k(h2_hbm, ids_hbm, zf_hbm, acc_out,
          blk, rows_a, rows_b, acc_sh, sem_a, sem_b):
        cid = lax.axis_index("c")
        sid = lax.axis_index("s")
        r0 = sid * stripe
        zf_cp = pltpu.async_copy(zf_hbm.at[pl.ds(r0, stripe)],
                                 acc_sh.at[pl.ds(r0, stripe)], sem_a)
        zf_cp.wait()
        plsc.subcore_barrier()

        def gather(j, rows, sem):
            return pltpu.async_copy(h2_hbm.at[blk.at[j, 0]], rows, sem)

        def scatter(j, rows):
            pltpu.sync_copy(rows, acc_sh.at[blk.at[j, 1]], add=True)

        def sb_body(s, carry):
            pltpu.sync_copy(ids_hbm.at[sid, pl.ds(s * sb, sb)], blk)

            def tbody(j, c2):
                for i in range(chunk // LANES):
                    sl = pl.ds(i * LANES, LANES)
                    blk[j, 0, sl] = blk[j, 0, sl] * 2 + cid
                return c2

            lax.fori_loop(0, sb, tbody, 0)
            gather(0, rows_a, sem_a)

            def fbody(p, c2):
                j0 = 2 * p
                gather(j0 + 1, rows_b, sem_b)
                pltpu.make_async_copy(h2_hbm.at[blk.at[j0, 0]],
                                      rows_a, sem_a).wait()
                scatter(j0, rows_a)

                @pl.when(j0 + 2 < sb)
                def _():
                    gather(j0 + 2, rows_a, sem_a)

                pltpu.make_async_copy(h2_hbm.at[blk.at[j0 + 1, 0]],
                                      rows_b, sem_b).wait()
                scatter(j0 + 1, rows_b)
                return c2

            lax.fori_loop(0, sb // 2, fbody, 0)
            return carry

        lax.fori_loop(0, n_sb, sb_body, 0)
        plsc.subcore_barrier()
        pltpu.sync_copy(acc_sh.at[pl.ds(r0, stripe)],
                        acc_out.at[cid, pl.ds(r0, stripe)])

    return k(h2, ids, zeros_f)


def _pack_ids(src, dst, chunk):
    e_tile = src.shape[0] // NS
    n_chunks = e_tile // chunk
    s = src.astype(jnp.int32).reshape(NS, n_chunks, 1, chunk)
    d = dst.astype(jnp.int32).reshape(NS, n_chunks, 1, chunk)
    return jnp.concatenate([s, d], axis=2)


def _tc_sage(h_prev, acc, w_self, w_neigh, b, relu, append_ones, n_out, bm):
    F = h_prev.shape[1]
    fw = acc.shape[2]
    H = w_self.shape[1]
    Hh = H // 2
    out_w = H + 32 if append_ones else H

    def body(hd_ref, a_ref, ws_ref, wn_ref, b_ref, o_ref):
        cnt = jnp.maximum(a_ref[0, :, fw - 16:fw - 15], 1.0)
        neigh = jnp.dot(a_ref[0], wn_ref[0], preferred_element_type=jnp.float32)
        neigh += jnp.dot(a_ref[1], wn_ref[1], preferred_element_type=jnp.float32)
        out = jnp.dot(hd_ref[...], ws_ref[...],
                      preferred_element_type=jnp.float32)
        out += neigh / cnt
        out += b_ref[...]
        if relu:
            out = jnp.maximum(out, 0.0)
        if append_ones:
            ones = jnp.ones((out.shape[0], 16), jnp.float32)
            out = jnp.concatenate(
                [out[:, :Hh], ones, out[:, Hh:], ones], axis=1)
        o_ref[...] = out

    return pl.pallas_call(
        body,
        grid=(n_out // bm,),
        in_specs=[
            pl.BlockSpec((bm, F), lambda i: (i, 0)),
            pl.BlockSpec((2, bm, fw), lambda i: (0, i, 0)),
            pl.BlockSpec((F, H), lambda i: (0, 0)),
            pl.BlockSpec((2, fw, H), lambda i: (0, 0, 0)),
            pl.BlockSpec((1, H), lambda i: (0, 0)),
        ],
        out_specs=pl.BlockSpec((bm, out_w), lambda i: (i, 0)),
        out_shape=jax.ShapeDtypeStruct((n_out, out_w), jnp.float32),
    )(h_prev, acc, w_self, w_neigh, b)


def _pad_neigh(w_neigh):
    F, H = w_neigh.shape
    wn = w_neigh.reshape(2, F // 2, H)
    return jnp.concatenate([wn, jnp.zeros((2, 16, H), w_neigh.dtype)], axis=1)


def _pad_self(w_self):
    return _pad_neigh(w_self).reshape(-1, w_self.shape[1])


def kernel(x, src1, dst1, src2, dst2, src3, dst3, n_dst1, n_dst2, n_dst3,
           W_self1, W_neigh1, b1, W_self2, W_neigh2, b2,
           W_self3, W_neigh3, b3):
    N1, N2, N3 = 20000, 8000, 4096
    zero = ((jnp.asarray(n_dst1) - N1)
            + (jnp.asarray(n_dst2) - N2)
            + (jnp.asarray(n_dst3) - N3)).astype(x.dtype)

    xr = x.reshape(-1, 2, 64)
    x_aug = jnp.concatenate(
        [xr, jnp.ones((xr.shape[0], 2, 16), x.dtype)], axis=2).reshape(-1, 80)

    acc1 = _sc_mean_agg(x_aug, _pack_ids(src1, dst1, 80), N1,
                        fw=80, chunk=80, sb=50)
    h1 = _tc_sage(x, acc1, W_self1, _pad_neigh(W_neigh1), b1.reshape(1, 256),
                  relu=True, append_ones=True, n_out=N1, bm=400)

    acc2 = _sc_mean_agg(h1.reshape(-1, 144), _pack_ids(src2, dst2, 80), N2,
                        fw=144, chunk=80, sb=100)
    h2 = _tc_sage(h1, acc2, _pad_self(W_self2), _pad_neigh(W_neigh2),
                  b2.reshape(1, 256), relu=True, append_ones=True,
                  n_out=N2, bm=400)

    acc3 = _sc_mean_agg(h2.reshape(-1, 144), _pack_ids(src3, dst3, 128), N3,
                        fw=144, chunk=128, sb=32)
    h3 = _tc_sage(h2, acc3, _pad_self(W_self3), _pad_neigh(W_neigh3),
                  (b3 + zero).reshape(1, 256), relu=False, append_ones=False,
                  n_out=N3, bm=512)
    return h3


# speedup vs baseline: 7.2493x; 1.2589x over previous
import functools

import jax
import jax.numpy as jnp
from jax import lax
from jax.experimental import pallas as pl
from jax.experimental.pallas import tpu as pltpu
from jax.experimental.pallas import tpu_sc as plsc

NC = 2
NS = 16
LANES = 16


def _sc_mean_agg(h2, src, dst, n_src, n_dst, fh, chunk, sb):
    E = src.shape[0]
    e_tile = E // NS
    n_chunks = e_tile // chunk
    n_sb = n_chunks // sb
    stripe = -(-n_dst // NS)
    stripe += (-stripe) % 8
    n_pad = stripe * NS
    assert e_tile % chunk == 0 and n_chunks % sb == 0
    assert sb % 2 == 0 and chunk % 8 == 0 and chunk <= 128

    src3 = src.astype(jnp.int32).reshape(NS, n_chunks, chunk)
    dst3 = dst.astype(jnp.int32).reshape(NS, n_chunks, chunk)

    mesh = plsc.VectorSubcoreMesh(core_axis_name="c", subcore_axis_name="s",
                                  num_cores=NC, num_subcores=NS)
    zeros_f = jnp.zeros((n_pad, fh), jnp.float32)
    zeros_c = jnp.zeros((n_pad, 16), jnp.float32)
    ones_c = jnp.ones((chunk, 16), jnp.float32)

    @functools.partial(
        pl.kernel,
        out_type=(jax.ShapeDtypeStruct((NC, n_pad, fh), jnp.float32),
                  jax.ShapeDtypeStruct((NC, n_pad, 16), jnp.float32)),
        mesh=mesh,
        scratch_types=[
            pltpu.VMEM((sb, chunk), jnp.int32),
            pltpu.VMEM((sb, chunk), jnp.int32),
            pltpu.VMEM((chunk, fh), jnp.float32),
            pltpu.VMEM((chunk, fh), jnp.float32),
            pltpu.VMEM((chunk, 16), jnp.float32),
            pltpu.VMEM_SHARED((n_pad, fh), jnp.float32),
            pltpu.VMEM_SHARED((n_pad, 16), jnp.float32),
            pltpu.SemaphoreType.DMA,
            pltpu.SemaphoreType.DMA,
            pltpu.SemaphoreType.DMA,
        ],
        compiler_params=pltpu.CompilerParams(use_tc_tiling_on_sc=False),
    )
    def k(h2_hbm, src_hbm, dst_hbm, zf_hbm, zc_hbm, ones_hbm,
          agg_out, cnt_out,
          blk_s, blk_d, rows_a, rows_b, ones_v, acc_sh, cnt_sh,
          sem_a, sem_b, sem_c):
        cid = lax.axis_index("c")
        sid = lax.axis_index("s")
        r0 = sid * stripe
        zf_cp = pltpu.async_copy(zf_hbm.at[pl.ds(r0, stripe)],
                                 acc_sh.at[pl.ds(r0, stripe)], sem_a)
        zc_cp = pltpu.async_copy(zc_hbm.at[pl.ds(r0, stripe)],
                                 cnt_sh.at[pl.ds(r0, stripe)], sem_b)
        pltpu.sync_copy(ones_hbm, ones_v)
        zf_cp.wait()
        zc_cp.wait()
        plsc.subcore_barrier()

        base = cid * n_src
        half = sb // 2

        def gather(j, rows, sem):
            return pltpu.async_copy(h2_hbm.at[blk_s.at[j]], rows, sem)

        def sb_body(s, carry):
            s_cp = pltpu.async_copy(src_hbm.at[sid, pl.ds(s * sb, sb)],
                                    blk_s, sem_a)
            pltpu.async_copy(dst_hbm.at[sid, pl.ds(s * sb, sb)],
                             blk_d, sem_b).wait()
            s_cp.wait()

            def tbody(j, c2):
                for i in range(chunk // LANES):
                    sl = pl.ds(i * LANES, LANES)
                    blk_s[j, sl] = blk_s[j, sl] + base
                return c2

            lax.fori_loop(0, sb, tbody, 0)
            gather(0, rows_a, sem_a)

            def fbody(p, c2):
                j0 = 2 * p
                gather(j0 + 1, rows_b, sem_b)

                @pl.when((j0 < half) == (cid == 0))
                def _():
                    pltpu.async_copy(ones_v, cnt_sh.at[blk_d.at[j0]],
                                     sem_c, add=True)

                pltpu.make_async_copy(h2_hbm.at[blk_s.at[j0]],
                                      rows_a, sem_a).wait()
                pltpu.sync_copy(rows_a, acc_sh.at[blk_d.at[j0]], add=True)

                @pl.when(j0 + 2 < sb)
                def _():
                    gather(j0 + 2, rows_a, sem_a)

                @pl.when((j0 + 1 < half) == (cid == 0))
                def _():
                    pltpu.async_copy(ones_v, cnt_sh.at[blk_d.at[j0 + 1]],
                                     sem_c, add=True)

                pltpu.make_async_copy(h2_hbm.at[blk_s.at[j0 + 1]],
                                      rows_b, sem_b).wait()
                pltpu.sync_copy(rows_b, acc_sh.at[blk_d.at[j0 + 1]], add=True)
                return c2

            lax.fori_loop(0, sb // 2, fbody, 0)

            def dbody(j, c2):
                pltpu.make_async_copy(ones_hbm, ones_v, sem_c).wait()
                return c2

            lax.fori_loop(0, half, dbody, 0)
            return carry

        lax.fori_loop(0, n_sb, sb_body, 0)
        plsc.subcore_barrier()
        pltpu.sync_copy(acc_sh.at[pl.ds(r0, stripe)],
                        agg_out.at[cid, pl.ds(r0, stripe)])
        pltpu.sync_copy(cnt_sh.at[pl.ds(r0, stripe)],
                        cnt_out.at[cid, pl.ds(r0, stripe)])

    return k(h2, src3, dst3, zeros_f, zeros_c, ones_c)


def _tc_sage(h_prev, agg, cnt, w_self, w_neigh, b, relu, split_out, n_out, bm):
    stacked_in = h_prev.ndim == 3
    fh = agg.shape[2]
    H = w_neigh.shape[2]
    Hh = H // 2

    def body(hd_ref, a_ref, c_ref, ws_ref, wn_ref, b_ref, o_ref):
        c = jnp.maximum(c_ref[0, :, 0:1] + c_ref[1, :, 0:1], 1.0)
        neigh = jnp.dot(a_ref[0], wn_ref[0], preferred_element_type=jnp.float32)
        neigh += jnp.dot(a_ref[1], wn_ref[1], preferred_element_type=jnp.float32)
        if stacked_in:
            out = jnp.dot(hd_ref[0], ws_ref[0],
                          preferred_element_type=jnp.float32)
            out += jnp.dot(hd_ref[1], ws_ref[1],
                           preferred_element_type=jnp.float32)
        else:
            out = jnp.dot(hd_ref[...], ws_ref[...],
                          preferred_element_type=jnp.float32)
        out += neigh / c
        out += b_ref[...]
        if relu:
            out = jnp.maximum(out, 0.0)
        if split_out:
            o_ref[0] = out[:, :Hh]
            o_ref[1] = out[:, Hh:]
        else:
            o_ref[...] = out

    if stacked_in:
        hd_spec = pl.BlockSpec((2, bm, h_prev.shape[2]), lambda i: (0, i, 0))
        ws_spec = pl.BlockSpec(w_self.shape, lambda i: (0, 0, 0))
    else:
        hd_spec = pl.BlockSpec((bm, h_prev.shape[1]), lambda i: (i, 0))
        ws_spec = pl.BlockSpec(w_self.shape, lambda i: (0, 0))
    if split_out:
        out_spec = pl.BlockSpec((2, bm, Hh), lambda i: (0, i, 0))
        out_shape = jax.ShapeDtypeStruct((2, n_out, Hh), jnp.float32)
    else:
        out_spec = pl.BlockSpec((bm, H), lambda i: (i, 0))
        out_shape = jax.ShapeDtypeStruct((n_out, H), jnp.float32)

    return pl.pallas_call(
        body,
        grid=(n_out // bm,),
        in_specs=[
            hd_spec,
            pl.BlockSpec((2, bm, fh), lambda i: (0, i, 0)),
            pl.BlockSpec((2, bm, 16), lambda i: (0, i, 0)),
            ws_spec,
            pl.BlockSpec((2, fh, H), lambda i: (0, 0, 0)),
            pl.BlockSpec((1, H), lambda i: (0, 0)),
        ],
        out_specs=out_spec,
        out_shape=out_shape,
    )(h_prev, agg, cnt, w_self, w_neigh, b)


def kernel(x, src1, dst1, src2, dst2, src3, dst3, n_dst1, n_dst2, n_dst3,
           W_self1, W_neigh1, b1, W_self2, W_neigh2, b2,
           W_self3, W_neigh3, b3):
    N1, N2, N3 = 20000, 8000, 4096
    n_src = x.shape[0]
    zero = ((jnp.asarray(n_dst1) - N1)
            + (jnp.asarray(n_dst2) - N2)
            + (jnp.asarray(n_dst3) - N3)).astype(x.dtype)

    xh = x.reshape(n_src, 2, 64).transpose(1, 0, 2).reshape(2 * n_src, 64)

    agg1, cnt1 = _sc_mean_agg(xh, src1, dst1, n_src, N1,
                              fh=64, chunk=80, sb=50)
    h1 = _tc_sage(x, agg1, cnt1, W_self1, W_neigh1.reshape(2, 64, 256),
                  b1.reshape(1, 256), relu=True, split_out=True,
                  n_out=N1, bm=400)

    agg2, cnt2 = _sc_mean_agg(h1.reshape(2 * N1, 128), src2, dst2, N1, N2,
                              fh=128, chunk=80, sb=100)
    h2 = _tc_sage(h1, agg2, cnt2, W_self2.reshape(2, 128, 256),
                  W_neigh2.reshape(2, 128, 256), b2.reshape(1, 256),
                  relu=True, split_out=True, n_out=N2, bm=400)

    agg3, cnt3 = _sc_mean_agg(h2.reshape(2 * N2, 128), src3, dst3, N2, N3,
                              fh=128, chunk=128, sb=32)
    h3 = _tc_sage(h2, agg3, cnt3, W_self3.reshape(2, 128, 256),
                  W_neigh3.reshape(2, 128, 256), (b3 + zero).reshape(1, 256),
                  relu=False, split_out=False, n_out=N3, bm=512)
    return h3


# speedup vs baseline: 9.1243x; 1.2586x over previous
import functools

import jax
import jax.numpy as jnp
from jax import lax
from jax.experimental import pallas as pl
from jax.experimental.pallas import tpu as pltpu
from jax.experimental.pallas import tpu_sc as plsc

NC = 2
NS = 16
LANES = 16


def _sc_mean_agg(h2, src, dst, n_src, n_dst, fh, chunk, sb, interleave=False):
    E = src.shape[0]
    e_tile = E // NS
    n_chunks = e_tile // chunk
    n_sb = n_chunks // sb
    stripe = -(-n_dst // NS)
    stripe += (-stripe) % 8
    n_pad = stripe * NS
    assert e_tile % chunk == 0 and n_chunks % sb == 0
    assert sb % 2 == 0 and chunk % 8 == 0 and chunk <= 128

    src3 = src.astype(jnp.int32).reshape(NS, n_chunks, chunk)
    dst3 = dst.astype(jnp.int32).reshape(NS, n_chunks, chunk)

    mesh = plsc.VectorSubcoreMesh(core_axis_name="c", subcore_axis_name="s",
                                  num_cores=NC, num_subcores=NS)
    zeros_f = jnp.zeros((n_pad, fh), jnp.float32)
    zeros_c = jnp.zeros((n_pad, 16), jnp.float32)
    ones_c = jnp.ones((chunk, 16), jnp.float32)

    @functools.partial(
        pl.kernel,
        out_type=(jax.ShapeDtypeStruct((NC, n_pad, fh), jnp.float32),
                  jax.ShapeDtypeStruct((NC, n_pad, 16), jnp.float32)),
        mesh=mesh,
        scratch_types=[
            pltpu.VMEM((sb, chunk), jnp.int32),
            pltpu.VMEM((sb, chunk), jnp.int32),
            pltpu.VMEM((chunk, fh), jnp.float32),
            pltpu.VMEM((chunk, fh), jnp.float32),
            pltpu.VMEM((chunk, 16), jnp.float32),
            pltpu.VMEM_SHARED((n_pad, fh), jnp.float32),
            pltpu.VMEM_SHARED((n_pad, 16), jnp.float32),
            pltpu.SemaphoreType.DMA,
            pltpu.SemaphoreType.DMA,
            pltpu.SemaphoreType.DMA,
        ],
        compiler_params=pltpu.CompilerParams(use_tc_tiling_on_sc=False),
    )
    def k(h2_hbm, src_hbm, dst_hbm, zf_hbm, zc_hbm, ones_hbm,
          agg_out, cnt_out,
          blk_s, blk_d, rows_a, rows_b, ones_v, acc_sh, cnt_sh,
          sem_a, sem_b, sem_c):
        cid = lax.axis_index("c")
        sid = lax.axis_index("s")
        r0 = sid * stripe
        zf_cp = pltpu.async_copy(zf_hbm.at[pl.ds(r0, stripe)],
                                 acc_sh.at[pl.ds(r0, stripe)], sem_a)
        zc_cp = pltpu.async_copy(zc_hbm.at[pl.ds(r0, stripe)],
                                 cnt_sh.at[pl.ds(r0, stripe)], sem_b)
        pltpu.sync_copy(ones_hbm, ones_v)
        zf_cp.wait()
        zc_cp.wait()
        plsc.subcore_barrier()

        base = cid if interleave else cid * n_src
        half = sb // 2

        def gather(j, rows, sem):
            return pltpu.async_copy(h2_hbm.at[blk_s.at[j]], rows, sem)

        def sb_body(s, carry):
            s_cp = pltpu.async_copy(src_hbm.at[sid, pl.ds(s * sb, sb)],
                                    blk_s, sem_a)
            pltpu.async_copy(dst_hbm.at[sid, pl.ds(s * sb, sb)],
                             blk_d, sem_b).wait()
            s_cp.wait()

            def tbody(j, c2):
                for i in range(chunk // LANES):
                    sl = pl.ds(i * LANES, LANES)
                    if interleave:
                        blk_s[j, sl] = blk_s[j, sl] * 2 + base
                    else:
                        blk_s[j, sl] = blk_s[j, sl] + base
                return c2

            lax.fori_loop(0, sb, tbody, 0)
            gather(0, rows_a, sem_a)

            def fbody(p, c2):
                j0 = 2 * p
                gather(j0 + 1, rows_b, sem_b)

                @pl.when((j0 < half) == (cid == 0))
                def _():
                    pltpu.async_copy(ones_v, cnt_sh.at[blk_d.at[j0]],
                                     sem_c, add=True)

                pltpu.make_async_copy(h2_hbm.at[blk_s.at[j0]],
                                      rows_a, sem_a).wait()
                pltpu.sync_copy(rows_a, acc_sh.at[blk_d.at[j0]], add=True)

                @pl.when(j0 + 2 < sb)
                def _():
                    gather(j0 + 2, rows_a, sem_a)

                @pl.when((j0 + 1 < half) == (cid == 0))
                def _():
                    pltpu.async_copy(ones_v, cnt_sh.at[blk_d.at[j0 + 1]],
                                     sem_c, add=True)

                pltpu.make_async_copy(h2_hbm.at[blk_s.at[j0 + 1]],
                                      rows_b, sem_b).wait()
                pltpu.sync_copy(rows_b, acc_sh.at[blk_d.at[j0 + 1]], add=True)
                return c2

            lax.fori_loop(0, sb // 2, fbody, 0)

            def dbody(j, c2):
                pltpu.make_async_copy(ones_hbm, ones_v, sem_c).wait()
                return c2

            lax.fori_loop(0, half, dbody, 0)
            return carry

        lax.fori_loop(0, n_sb, sb_body, 0)
        plsc.subcore_barrier()
        pltpu.sync_copy(acc_sh.at[pl.ds(r0, stripe)],
                        agg_out.at[cid, pl.ds(r0, stripe)])
        pltpu.sync_copy(cnt_sh.at[pl.ds(r0, stripe)],
                        cnt_out.at[cid, pl.ds(r0, stripe)])

    return k(h2, src3, dst3, zeros_f, zeros_c, ones_c)


def _tc_sage(h_prev, agg, cnt, w_self, w_neigh, b, relu, split_out, n_out, bm):
    stacked_in = h_prev.ndim == 3
    fh = agg.shape[2]
    H = w_neigh.shape[2]
    Hh = H // 2

    def body(hd_ref, a_ref, c_ref, ws_ref, wn_ref, b_ref, o_ref):
        c = jnp.maximum(c_ref[0, :, 0:1] + c_ref[1, :, 0:1], 1.0)
        neigh = jnp.dot(a_ref[0], wn_ref[0], preferred_element_type=jnp.float32)
        neigh += jnp.dot(a_ref[1], wn_ref[1], preferred_element_type=jnp.float32)
        if stacked_in:
            out = jnp.dot(hd_ref[0], ws_ref[0],
                          preferred_element_type=jnp.float32)
            out += jnp.dot(hd_ref[1], ws_ref[1],
                           preferred_element_type=jnp.float32)
        else:
            out = jnp.dot(hd_ref[...], ws_ref[...],
                          preferred_element_type=jnp.float32)
        out += neigh / c
        out += b_ref[...]
        if relu:
            out = jnp.maximum(out, 0.0)
        if split_out:
            o_ref[0] = out[:, :Hh]
            o_ref[1] = out[:, Hh:]
        else:
            o_ref[...] = out

    if stacked_in:
        hd_spec = pl.BlockSpec((2, bm, h_prev.shape[2]), lambda i: (0, i, 0))
        ws_spec = pl.BlockSpec(w_self.shape, lambda i: (0, 0, 0))
    else:
        hd_spec = pl.BlockSpec((bm, h_prev.shape[1]), lambda i: (i, 0))
        ws_spec = pl.BlockSpec(w_self.shape, lambda i: (0, 0))
    if split_out:
        out_spec = pl.BlockSpec((2, bm, Hh), lambda i: (0, i, 0))
        out_shape = jax.ShapeDtypeStruct((2, n_out, Hh), jnp.float32)
    else:
        out_spec = pl.BlockSpec((bm, H), lambda i: (i, 0))
        out_shape = jax.ShapeDtypeStruct((n_out, H), jnp.float32)

    return pl.pallas_call(
        body,
        grid=(n_out // bm,),
        in_specs=[
            hd_spec,
            pl.BlockSpec((2, bm, fh), lambda i: (0, i, 0)),
            pl.BlockSpec((2, bm, 16), lambda i: (0, i, 0)),
            ws_spec,
            pl.BlockSpec((2, fh, H), lambda i: (0, 0, 0)),
            pl.BlockSpec((1, H), lambda i: (0, 0)),
        ],
        out_specs=out_spec,
        out_shape=out_shape,
    )(h_prev, agg, cnt, w_self, w_neigh, b)


def kernel(x, src1, dst1, src2, dst2, src3, dst3, n_dst1, n_dst2, n_dst3,
           W_self1, W_neigh1, b1, W_self2, W_neigh2, b2,
           W_self3, W_neigh3, b3):
    N1, N2, N3 = 20000, 8000, 4096
    n_src = x.shape[0]
    zero = ((jnp.asarray(n_dst1) - N1)
            + (jnp.asarray(n_dst2) - N2)
            + (jnp.asarray(n_dst3) - N3)).astype(x.dtype)

    agg1, cnt1 = _sc_mean_agg(x.reshape(2 * n_src, 64), src1, dst1, n_src, N1,
                              fh=64, chunk=80, sb=50, interleave=True)
    h1 = _tc_sage(x, agg1, cnt1, W_self1, W_neigh1.reshape(2, 64, 256),
                  b1.reshape(1, 256), relu=True, split_out=True,
                  n_out=N1, bm=800)

    agg2, cnt2 = _sc_mean_agg(h1.reshape(2 * N1, 128), src2, dst2, N1, N2,
                              fh=128, chunk=80, sb=100)
    h2 = _tc_sage(h1, agg2, cnt2, W_self2.reshape(2, 128, 256),
                  W_neigh2.reshape(2, 128, 256), b2.reshape(1, 256),
                  relu=True, split_out=True, n_out=N2, bm=800)

    agg3, cnt3 = _sc_mean_agg(h2.reshape(2 * N2, 128), src3, dst3, N2, N3,
                              fh=128, chunk=128, sb=32)
    h3 = _tc_sage(h2, agg3, cnt3, W_self3.reshape(2, 128, 256),
                  W_neigh3.reshape(2, 128, 256), (b3 + zero).reshape(1, 256),
                  relu=False, split_out=False, n_out=N3, bm=1024)
    return h3
